# Initial kernel scaffold; baseline (speedup 1.0000x reference)
import jax
import jax.numpy as jnp
from jax.experimental import pallas as pl


def kernel(input_x, char_emb, weight_char_emb, fc1_w, fc1_b):
    raise NotImplementedError("write your pallas kernel here")


# speedup vs baseline: 288.9624x; 288.9624x over previous
import functools

import jax
import jax.numpy as jnp
from jax import lax
from jax.experimental import pallas as pl
from jax.experimental.pallas import tpu as pltpu
from jax.experimental.pallas import tpu_sc as plsc

_LANES = 16
_NUM_CORES = 2
_NUM_SUBCORES = 16
_VOCAB_PAD = 1024


def _vtable_tc_kernel(emb_ref, fcw_ref, out_ref):
    v = lax.dot_general(
        emb_ref[...], fcw_ref[...],
        (((1,), (1,)), ((), ())),
        preferred_element_type=jnp.float32,
    )
    out_ref[...] = jnp.zeros_like(out_ref)
    out_ref[0:emb_ref.shape[0], :] = v


def kernel(input_x, char_emb, weight_char_emb, fc1_w, fc1_b):
    B, M = input_x.shape
    V, E = char_emb.shape
    NW = _NUM_CORES * _NUM_SUBCORES
    BPW = B // NW

    v_tab = pl.pallas_call(
        _vtable_tc_kernel,
        out_shape=jax.ShapeDtypeStruct((_VOCAB_PAD, 1), jnp.float32),
    )(char_emb, fc1_w)
    v_tab = v_tab.reshape(_VOCAB_PAD)

    aux = jnp.concatenate([
        weight_char_emb,
        jnp.zeros((112 - M,), jnp.float32),
        fc1_b,
        jnp.zeros((15,), jnp.float32),
    ])

    xt = input_x.T

    mesh = plsc.VectorSubcoreMesh(core_axis_name="c", subcore_axis_name="s")

    @functools.partial(
        pl.kernel,
        out_type=jax.ShapeDtypeStruct((B,), jnp.float32),
        mesh=mesh,
        compiler_params=pltpu.CompilerParams(needs_layout_passes=False),
        scratch_types=[
            pltpu.VMEM((M, BPW), jnp.int32),
            pltpu.VMEM((_VOCAB_PAD,), jnp.float32),
            pltpu.VMEM((128,), jnp.float32),
            pltpu.VMEM((BPW,), jnp.float32),
        ],
    )
    def sc_score(xt_hbm, v_hbm, aux_hbm, out_hbm, x_v, v_v, aux_v, o_v):
        wid = lax.axis_index("s") * _NUM_CORES + lax.axis_index("c")
        base = wid * BPW
        pltpu.sync_copy(v_hbm, v_v)
        pltpu.sync_copy(aux_hbm, aux_v)
        pltpu.sync_copy(xt_hbm.at[:, pl.ds(base, BPW)], x_v)
        bias = aux_v[pl.ds(112, _LANES)][0]

        def g_body(g, carry):
            gb = g * _LANES

            def m_body(m, acc):
                idx = x_v[m, pl.ds(gb, _LANES)]
                gv = plsc.load_gather(v_v, [idx])
                w_m = aux_v[pl.ds(m, _LANES)][0]
                return acc + gv * w_m

            acc = lax.fori_loop(
                0, M, m_body, jnp.zeros((_LANES,), jnp.float32), unroll=4)
            o_v[pl.ds(gb, _LANES)] = acc + bias
            return carry

        lax.fori_loop(0, BPW // _LANES, g_body, 0)
        pltpu.sync_copy(o_v, out_hbm.at[pl.ds(base, BPW)])

    return sc_score(xt, v_tab, aux)
